# baseline (speedup 1.0000x reference)
import functools

import jax
import jax.numpy as jnp
from jax import lax
from jax.experimental import pallas as pl
from jax.experimental.pallas import tpu as pltpu
from jax.experimental.pallas import tpu_sc as plsc

EMB = 768
HEADS = 8
K = 128
GADD = 8
RADD = 8
REGION = 128
MIN_SIGMA = 0.05
SIGMA_BOOST = 2.0
EPS = 1e-7
T = 2048
NPTS = 4 + RADD + GADD
VS = K * NPTS
SCALE = float(EMB) ** -0.25

NW = 32
CPW = VS // NW


def _prep_kernel(m_ref, s_ref, rrx_ref, rry_ref, rgx_ref, rgy_ref,
                 rows_ref, cols_ref, mt_ref, st_ref):
    t = float(T)
    mtx = jax.nn.sigmoid(m_ref[0:1, :]) * (t - 1.0)
    mty = jax.nn.sigmoid(m_ref[1:2, :]) * (t - 1.0)
    mt_ref[0:1, :] = mtx
    mt_ref[1:2, :] = mty
    s = s_ref[0:1, :] + SIGMA_BOOST
    sp = jnp.maximum(s, 0.0) + jnp.log1p(jnp.exp(-jnp.abs(s)))
    st_ref[0:1, :] = sp + MIN_SIGMA

    flx = jnp.floor(mtx)
    fly = jnp.floor(mty)
    sub4i = lax.broadcasted_iota(jnp.int32, (4, 1), 0)
    offx = (sub4i // 2).astype(jnp.float32)
    offy = (sub4i % 2).astype(jnp.float32)
    cornx = flx + offx
    corny = fly + offy
    lox = jnp.clip(flx - REGION // 2, 0.0, t - REGION)
    loy = jnp.clip(fly - REGION // 2, 0.0, t - REGION)
    relx = lox + rrx_ref[...]
    rely = loy + rry_ref[...]
    globx = rgx_ref[...]
    globy = rgy_ref[...]
    px = jnp.concatenate([cornx, relx, globx], axis=0)
    py = jnp.concatenate([corny, rely, globy], axis=0)
    px = jnp.clip(px, 0.0, t - 1.0)
    py = jnp.clip(py, 0.0, t - 1.0)
    rows_ref[...] = px.astype(jnp.int32)
    cols_ref[...] = py.astype(jnp.int32)


_DUP_CHUNK = 512


def _weights_kernel(rows_cm_ref, cols_cm_ref, rows_rm_ref, cols_rm_ref,
                    mt_ref, st_ref, w_ref):
    t = T
    rows_cm = rows_cm_ref[...]
    cols_cm = cols_cm_ref[...]
    codes_r = rows_cm * t + cols_cm
    codes_c = rows_rm_ref[...] * t + cols_rm_ref[...]
    row_idx = lax.broadcasted_iota(jnp.int32, (VS, 1), 0)
    dup = jnp.zeros((VS, 1), dtype=jnp.bool_)
    for c in range(VS // _DUP_CHUNK):
        cc = codes_c[:, c * _DUP_CHUNK:(c + 1) * _DUP_CHUNK]
        col_idx = lax.broadcasted_iota(jnp.int32, (1, _DUP_CHUNK), 1) + c * _DUP_CHUNK
        eq = (codes_r == cc) & (col_idx < row_idx)
        dup = dup | jnp.any(eq, axis=1, keepdims=True)
    ix = rows_cm.astype(jnp.float32)
    iy = cols_cm.astype(jnp.float32)
    mx = mt_ref[0:1, :]
    my = mt_ref[1:2, :]
    sg = st_ref[0:1, :] + EPS
    dx = (ix - mx) / sg
    dy = (iy - my) / sg
    props = jnp.exp(-0.5 * (dx * dx + dy * dy))
    props = props * (1.0 - dup.astype(jnp.float32))
    colsum = jnp.sum(props, axis=0, keepdims=True)
    props = props / (colsum + EPS)
    w_ref[...] = jnp.sum(props, axis=1, keepdims=True)


def _qkv_kernel(x_ref, w_ref, y_ref):
    j = pl.program_id(0)
    scale = jnp.where(j < 2 * HEADS, SCALE, 1.0)
    y_ref[0] = jnp.dot(x_ref[...], w_ref[...],
                       preferred_element_type=jnp.float32) * scale


_DCHUNK = 16
_NDCH = HEADS * CPW // _DCHUNK
_WIDX = HEADS * CPW
_VROWS = 64
_NVCH = _WIDX // _VROWS


def _sc_dots_body(y_hbm, idxq_hbm, idxk_hbm, idxv_hbm, dot_hbm, sv_hbm,
                  idxq_v, idxk_v, idxv_v, buf, tmp, dloc,
                  semq0, semq1, semk0, semk1, semw0, semw1):
    semqs = [semq0, semq1]
    semks = [semk0, semk1]
    semws = [semw0, semw1]
    wid = lax.axis_index("s") * 2 + lax.axis_index("c")
    base = pl.multiple_of(wid * _WIDX, 8)
    laneiota = lax.broadcasted_iota(jnp.int32, (16,), 0)
    pltpu.sync_copy(idxq_hbm.at[pl.ds(base, _WIDX)], idxq_v)
    pltpu.sync_copy(idxk_hbm.at[pl.ds(base, _WIDX)], idxk_v)
    pltpu.sync_copy(idxv_hbm.at[pl.ds(base, _WIDX)], idxv_v)
    bufsq = [buf.at[0, pl.ds(0, _DCHUNK)], buf.at[1, pl.ds(0, _DCHUNK)]]
    bufsk = [buf.at[0, pl.ds(_DCHUNK, _DCHUNK)],
             buf.at[1, pl.ds(_DCHUNK, _DCHUNK)]]

    def start(n, b):
        io = pl.multiple_of(n * _DCHUNK, 8)
        cq = pltpu.async_copy(y_hbm.at[idxq_v.at[pl.ds(io, _DCHUNK)]],
                              bufsq[b], semqs[b])
        ck = pltpu.async_copy(y_hbm.at[idxk_v.at[pl.ds(io, _DCHUNK)]],
                              bufsk[b], semks[b])
        return cq, ck

    def wait(b):
        pltpu.make_async_copy(y_hbm.at[idxq_v.at[pl.ds(0, _DCHUNK)]],
                              bufsq[b], semqs[b]).wait()
        pltpu.make_async_copy(y_hbm.at[idxk_v.at[pl.ds(0, _DCHUNK)]],
                              bufsk[b], semks[b]).wait()

    start(0, 0)

    def pair_body(np_, carry):
        for b in range(2):
            n = np_ * 2 + b

            @pl.when(n + 1 < _NDCH)
            def _():
                start(n + 1, 1 - b)

            wait(b)
            bq = bufsq[b]
            bk = bufsk[b]

            def row_body(r, rcarry):
                acc = bq[r, pl.ds(0, 16)] * bk[r, pl.ds(0, 16)]
                for j in range(1, EMB // 16):
                    acc = acc + bq[r, pl.ds(j * 16, 16)] * bk[r, pl.ds(j * 16, 16)]
                plsc.store_scatter(tmp, [laneiota + r * 16], acc)
                return rcarry

            lax.fori_loop(0, _DCHUNK, row_body, 0)
            dots16 = plsc.load_gather(tmp, [laneiota * 16])
            for cc in range(1, 16):
                dots16 = dots16 + plsc.load_gather(tmp, [laneiota * 16 + cc])
            dloc[pl.ds(pl.multiple_of(n * _DCHUNK, 8), _DCHUNK)] = dots16
        return carry

    lax.fori_loop(0, _NDCH // 2, pair_body, 0)
    pltpu.sync_copy(dloc, dot_hbm.at[pl.ds(base, _WIDX)])

    bufs = [buf.at[0], buf.at[1]]

    def start_g(n, b):
        io = pl.multiple_of(n * _VROWS, 8)
        pltpu.async_copy(y_hbm.at[idxv_v.at[pl.ds(io, _VROWS)]],
                         bufs[b], semqs[b])

    def wait_g(b):
        pltpu.make_async_copy(y_hbm.at[idxv_v.at[pl.ds(0, _VROWS)]],
                              bufs[b], semqs[b]).wait()

    def start_w(n, b):
        oo = pl.multiple_of(base + n * _VROWS, 8)
        pltpu.async_copy(bufs[b], sv_hbm.at[pl.ds(oo, _VROWS)], semws[b])

    def wait_w(b):
        pltpu.make_async_copy(bufs[b], sv_hbm.at[pl.ds(base, _VROWS)],
                              semws[b]).wait()

    start_g(0, 0)

    def vpair_body(np_, carry):
        for b in range(2):
            n = np_ * 2 + b

            @pl.when((n + 1 < _NVCH) & (n >= 1))
            def _():
                wait_w(1 - b)

            @pl.when(n + 1 < _NVCH)
            def _():
                start_g(n + 1, 1 - b)

            wait_g(b)
            start_w(n, b)
        return carry

    lax.fori_loop(0, _NVCH // 2, vpair_body, 0)
    wait_w(0)
    wait_w(1)


def _sc_sparse(y_flat, idxq, idxk, idxv):
    mesh = plsc.VectorSubcoreMesh(core_axis_name="c", subcore_axis_name="s")
    fn = functools.partial(
        pl.kernel,
        out_type=[
            jax.ShapeDtypeStruct((HEADS * VS,), jnp.float32),
            jax.ShapeDtypeStruct((VS * HEADS, EMB), jnp.float32),
        ],
        mesh=mesh,
        compiler_params=pltpu.CompilerParams(needs_layout_passes=False),
        scratch_types=[
            pltpu.VMEM((_WIDX,), jnp.int32),
            pltpu.VMEM((_WIDX,), jnp.int32),
            pltpu.VMEM((_WIDX,), jnp.int32),
            pltpu.VMEM((2, _VROWS, EMB), jnp.float32),
            pltpu.VMEM((_DCHUNK * 16,), jnp.float32),
            pltpu.VMEM((_WIDX,), jnp.float32),
            pltpu.SemaphoreType.DMA,
            pltpu.SemaphoreType.DMA,
            pltpu.SemaphoreType.DMA,
            pltpu.SemaphoreType.DMA,
            pltpu.SemaphoreType.DMA,
            pltpu.SemaphoreType.DMA,
        ],
    )(_sc_dots_body)
    return fn(y_flat, idxq, idxk, idxv)


_SM_CHUNK = 512


def _segmax_kernel(dot_ref, w_ref, rows_ref, mx_ref):
    h = pl.program_id(0)
    neg = jnp.float32(-jnp.inf)
    onehot = (lax.broadcasted_iota(jnp.int32, (1, HEADS), 1) == h
              ).astype(jnp.float32)
    laneid = lax.broadcasted_iota(jnp.int32, (1, T), 1)
    run = jnp.full((1, T), neg, jnp.float32)
    for c in range(VS // _SM_CHUNK):
        sl = pl.ds(c * _SM_CHUNK, _SM_CHUNK)
        logit = jnp.sum(dot_ref[sl, :] * onehot, axis=1, keepdims=True)
        logit = logit * w_ref[sl, :]
        mask = rows_ref[sl, :] == laneid
        vals = jnp.where(mask, logit, neg)
        run = jnp.maximum(run, jnp.max(vals, axis=0, keepdims=True))
    mx_ref[...] = run[None]


def _segsm_kernel(mx_ref, dot_ref, w_ref, rows_ref, p_ref):
    rt = (rows_ref[...] == lax.broadcasted_iota(jnp.int32, (1, T), 1)
          ).astype(jnp.float32)
    mrun = mx_ref[...]
    mrun = jnp.where(jnp.isfinite(mrun), mrun, 0.0)
    mx_cand = jnp.dot(rt, mrun, preferred_element_type=jnp.float32)
    logit = dot_ref[...] * w_ref[...]
    ex = jnp.exp(logit - mx_cand)
    srun = lax.dot_general(rt, ex, (((0,), (0,)), ((), ())),
                           preferred_element_type=jnp.float32)
    sm_cand = jnp.dot(rt, srun, preferred_element_type=jnp.float32)
    p_ref[...] = ex / (sm_cand + EPS)


_G_TILE = 256


def _contract_kernel(sv_ref, p_ref, e8_ref, wu_ref, g_ref):
    pexp = jnp.dot(p_ref[...], e8_ref[...], preferred_element_type=jnp.float32)
    g_ref[...] = jnp.dot(sv_ref[...] * pexp, wu_ref[...],
                         preferred_element_type=jnp.float32)


_H_TILE = 256


def _scatter_kernel(rows_ref, g_ref, bu_ref, o_ref):
    m = pl.program_id(0)
    rowiota = lax.broadcasted_iota(jnp.int32, (_H_TILE, 1), 0) + m * _H_TILE
    r = (rowiota == rows_ref[...]).astype(jnp.float32)
    o_ref[...] = jnp.dot(r, g_ref[...],
                         preferred_element_type=jnp.float32) + bu_ref[...]


def kernel(x, means, sigmas, Wq, Wk, Wv, Wu, bu):
    b, t, e = x.shape
    h = HEADS
    x2d = x.reshape(t, e)

    k1, k2 = jax.random.split(jax.random.key(42))
    rr = jax.random.randint(k1, (K, RADD, 2), 0, REGION).astype(jnp.float32)
    rg = jax.random.randint(k2, (K, GADD, 2), 0, t).astype(jnp.float32)

    means2 = means.T.reshape(2, K)
    sig2 = sigmas.reshape(1, K)
    rrx = rr[:, :, 0].T.reshape(RADD, K)
    rry = rr[:, :, 1].T.reshape(RADD, K)
    rgx = rg[:, :, 0].T.reshape(GADD, K)
    rgy = rg[:, :, 1].T.reshape(GADD, K)
    rows20, cols20, mt, st = pl.pallas_call(
        _prep_kernel,
        out_shape=[
            jax.ShapeDtypeStruct((NPTS, K), jnp.int32),
            jax.ShapeDtypeStruct((NPTS, K), jnp.int32),
            jax.ShapeDtypeStruct((2, K), jnp.float32),
            jax.ShapeDtypeStruct((1, K), jnp.float32),
        ],
    )(means2, sig2, rrx, rry, rgx, rgy)
    rows = rows20.T.reshape(VS)
    cols = cols20.T.reshape(VS)

    weights_cm = pl.pallas_call(
        _weights_kernel,
        out_shape=jax.ShapeDtypeStruct((VS, 1), jnp.float32),
    )(rows.reshape(VS, 1), cols.reshape(VS, 1),
      rows.reshape(1, VS), cols.reshape(1, VS), mt, st)

    wqkv = jnp.concatenate([Wq, Wk, Wv], axis=1)
    y = pl.pallas_call(
        _qkv_kernel,
        grid=(3 * h,),
        in_specs=[
            pl.BlockSpec((t, e), lambda j: (0, 0)),
            pl.BlockSpec((e, e), lambda j: (0, j)),
        ],
        out_specs=pl.BlockSpec((1, t, e), lambda j: (j, 0, 0)),
        out_shape=jax.ShapeDtypeStruct((3 * h, t, e), jnp.float32),
    )(x2d, wqkv)
    y_flat = y.reshape(3 * h * t, e)

    hoff = jnp.arange(h, dtype=jnp.int32) * t
    idxq = (rows.reshape(NW, 1, CPW) + hoff[None, :, None]).reshape(-1)
    idxk = (cols.reshape(NW, 1, CPW) + (h * t + hoff)[None, :, None]).reshape(-1)
    idxv = (cols[:, None] + (2 * h * t + hoff)[None, :]).reshape(VS * h)
    dots_wm, sv = _sc_sparse(y_flat, idxq, idxk, idxv)
    dots = dots_wm.reshape(NW, h, CPW).transpose(1, 0, 2).reshape(h, VS)
    sv2d = sv.reshape(VS, h * e)

    dots_cm = dots.T
    rows_cm = rows.reshape(VS, 1)
    mx = pl.pallas_call(
        _segmax_kernel,
        grid=(h,),
        in_specs=[
            pl.BlockSpec((VS, h), lambda hh: (0, 0)),
            pl.BlockSpec((VS, 1), lambda hh: (0, 0)),
            pl.BlockSpec((VS, 1), lambda hh: (0, 0)),
        ],
        out_specs=pl.BlockSpec((1, 1, T), lambda hh: (hh, 0, 0)),
        out_shape=jax.ShapeDtypeStruct((h, 1, T), jnp.float32),
    )(dots_cm, weights_cm, rows_cm)
    p_cm = pl.pallas_call(
        _segsm_kernel,
        out_shape=jax.ShapeDtypeStruct((VS, h), jnp.float32),
    )(mx.reshape(h, T).T, dots_cm, weights_cm, rows_cm)

    e8 = (jnp.repeat(jnp.eye(h, dtype=jnp.float32), e, axis=1)
          )
    g = pl.pallas_call(
        _contract_kernel,
        grid=(VS // _G_TILE,),
        in_specs=[
            pl.BlockSpec((_G_TILE, h * e), lambda m: (m, 0)),
            pl.BlockSpec((_G_TILE, h), lambda m: (m, 0)),
            pl.BlockSpec((h, h * e), lambda m: (0, 0)),
            pl.BlockSpec((h * e, e), lambda m: (0, 0)),
        ],
        out_specs=pl.BlockSpec((_G_TILE, e), lambda m: (m, 0)),
        out_shape=jax.ShapeDtypeStruct((VS, e), jnp.float32),
    )(sv2d, p_cm, e8, Wu)

    out = pl.pallas_call(
        _scatter_kernel,
        grid=(t // _H_TILE,),
        in_specs=[
            pl.BlockSpec((1, VS), lambda m: (0, 0)),
            pl.BlockSpec((VS, e), lambda m: (0, 0)),
            pl.BlockSpec((1, e), lambda m: (0, 0)),
        ],
        out_specs=pl.BlockSpec((_H_TILE, e), lambda m: (m, 0)),
        out_shape=jax.ShapeDtypeStruct((t, e), jnp.float32),
    )(rows.reshape(1, VS), g, bu.reshape(1, e))
    return out.reshape(b, t, e)


# speedup vs baseline: 1.0966x; 1.0966x over previous
import functools

import jax
import jax.numpy as jnp
from jax import lax
from jax.experimental import pallas as pl
from jax.experimental.pallas import tpu as pltpu
from jax.experimental.pallas import tpu_sc as plsc

EMB = 768
HEADS = 8
K = 128
GADD = 8
RADD = 8
REGION = 128
MIN_SIGMA = 0.05
SIGMA_BOOST = 2.0
EPS = 1e-7
T = 2048
NPTS = 4 + RADD + GADD
VS = K * NPTS
SCALE = float(EMB) ** -0.25

NW = 32
CPW = VS // NW


def _prep_kernel(m_ref, s_ref, rrx_ref, rry_ref, rgx_ref, rgy_ref,
                 rows_ref, cols_ref, mt_ref, st_ref):
    t = float(T)
    mtx = jax.nn.sigmoid(m_ref[0:1, :]) * (t - 1.0)
    mty = jax.nn.sigmoid(m_ref[1:2, :]) * (t - 1.0)
    mt_ref[0:1, :] = mtx
    mt_ref[1:2, :] = mty
    s = s_ref[0:1, :] + SIGMA_BOOST
    sp = jnp.maximum(s, 0.0) + jnp.log1p(jnp.exp(-jnp.abs(s)))
    st_ref[0:1, :] = sp + MIN_SIGMA

    flx = jnp.floor(mtx)
    fly = jnp.floor(mty)
    sub4i = lax.broadcasted_iota(jnp.int32, (4, 1), 0)
    offx = (sub4i // 2).astype(jnp.float32)
    offy = (sub4i % 2).astype(jnp.float32)
    cornx = flx + offx
    corny = fly + offy
    lox = jnp.clip(flx - REGION // 2, 0.0, t - REGION)
    loy = jnp.clip(fly - REGION // 2, 0.0, t - REGION)
    relx = lox + rrx_ref[...]
    rely = loy + rry_ref[...]
    globx = rgx_ref[...]
    globy = rgy_ref[...]
    px = jnp.concatenate([cornx, relx, globx], axis=0)
    py = jnp.concatenate([corny, rely, globy], axis=0)
    px = jnp.clip(px, 0.0, t - 1.0)
    py = jnp.clip(py, 0.0, t - 1.0)
    rows_ref[...] = px.astype(jnp.int32)
    cols_ref[...] = py.astype(jnp.int32)


_DUP_CHUNK = 512


def _weights_kernel(rows_cm_ref, cols_cm_ref, rows_rm_ref, cols_rm_ref,
                    mt_ref, st_ref, w_ref):
    t = T
    rows_cm = rows_cm_ref[...]
    cols_cm = cols_cm_ref[...]
    codes_r = rows_cm * t + cols_cm
    codes_c = rows_rm_ref[...] * t + cols_rm_ref[...]
    row_idx = lax.broadcasted_iota(jnp.int32, (VS, 1), 0)
    dup = jnp.zeros((VS, 1), dtype=jnp.bool_)
    for c in range(VS // _DUP_CHUNK):
        cc = codes_c[:, c * _DUP_CHUNK:(c + 1) * _DUP_CHUNK]
        col_idx = lax.broadcasted_iota(jnp.int32, (1, _DUP_CHUNK), 1) + c * _DUP_CHUNK
        eq = (codes_r == cc) & (col_idx < row_idx)
        dup = dup | jnp.any(eq, axis=1, keepdims=True)
    ix = rows_cm.astype(jnp.float32)
    iy = cols_cm.astype(jnp.float32)
    mx = mt_ref[0:1, :]
    my = mt_ref[1:2, :]
    sg = st_ref[0:1, :] + EPS
    dx = (ix - mx) / sg
    dy = (iy - my) / sg
    props = jnp.exp(-0.5 * (dx * dx + dy * dy))
    props = props * (1.0 - dup.astype(jnp.float32))
    colsum = jnp.sum(props, axis=0, keepdims=True)
    props = props / (colsum + EPS)
    w_ref[...] = jnp.sum(props, axis=1, keepdims=True)


def _qkv_kernel(x_ref, w_ref, y_ref):
    j = pl.program_id(0)
    scale = jnp.where(j < 2 * HEADS, SCALE, 1.0)
    y_ref[0] = jnp.dot(x_ref[...], w_ref[...],
                       preferred_element_type=jnp.float32) * scale


_DCHUNK = 16
_NDCH = HEADS * CPW // _DCHUNK
_WIDX = HEADS * CPW
_VROWS = 64
_NVCH = _WIDX // _VROWS


def _sc_dots_body(y_hbm, idxq_hbm, idxk_hbm, dot_hbm,
                  idxq_v, idxk_v, buf, tmp, dloc,
                  semq0, semq1, semk0, semk1):
    semqs = [semq0, semq1]
    semks = [semk0, semk1]
    wid = lax.axis_index("s") * 2 + lax.axis_index("c")
    base = pl.multiple_of(wid * _WIDX, 8)
    laneiota = lax.broadcasted_iota(jnp.int32, (16,), 0)
    pltpu.sync_copy(idxq_hbm.at[pl.ds(base, _WIDX)], idxq_v)
    pltpu.sync_copy(idxk_hbm.at[pl.ds(base, _WIDX)], idxk_v)
    bufsq = [buf.at[0, pl.ds(0, _DCHUNK)], buf.at[1, pl.ds(0, _DCHUNK)]]
    bufsk = [buf.at[0, pl.ds(_DCHUNK, _DCHUNK)],
             buf.at[1, pl.ds(_DCHUNK, _DCHUNK)]]

    def start(n, b):
        io = pl.multiple_of(n * _DCHUNK, 8)
        cq = pltpu.async_copy(y_hbm.at[idxq_v.at[pl.ds(io, _DCHUNK)]],
                              bufsq[b], semqs[b])
        ck = pltpu.async_copy(y_hbm.at[idxk_v.at[pl.ds(io, _DCHUNK)]],
                              bufsk[b], semks[b])
        return cq, ck

    def wait(b):
        pltpu.make_async_copy(y_hbm.at[idxq_v.at[pl.ds(0, _DCHUNK)]],
                              bufsq[b], semqs[b]).wait()
        pltpu.make_async_copy(y_hbm.at[idxk_v.at[pl.ds(0, _DCHUNK)]],
                              bufsk[b], semks[b]).wait()

    start(0, 0)

    def pair_body(np_, carry):
        for b in range(2):
            n = np_ * 2 + b

            @pl.when(n + 1 < _NDCH)
            def _():
                start(n + 1, 1 - b)

            wait(b)
            bq = bufsq[b]
            bk = bufsk[b]

            def row_body(r, rcarry):
                acc = bq[r, pl.ds(0, 16)] * bk[r, pl.ds(0, 16)]
                for j in range(1, EMB // 16):
                    acc = acc + bq[r, pl.ds(j * 16, 16)] * bk[r, pl.ds(j * 16, 16)]
                plsc.store_scatter(tmp, [laneiota + r * 16], acc)
                return rcarry

            lax.fori_loop(0, _DCHUNK, row_body, 0)
            dots16 = plsc.load_gather(tmp, [laneiota * 16])
            for cc in range(1, 16):
                dots16 = dots16 + plsc.load_gather(tmp, [laneiota * 16 + cc])
            dloc[pl.ds(pl.multiple_of(n * _DCHUNK, 8), _DCHUNK)] = dots16
        return carry

    lax.fori_loop(0, _NDCH // 2, pair_body, 0)
    pltpu.sync_copy(dloc, dot_hbm.at[pl.ds(base, _WIDX)])


def _sc_gatherv_body(y_hbm, idxv_hbm, sv_hbm, idxv_v, bufv,
                     semg0, semg1, semw0, semw1):
    semgs = [semg0, semg1]
    semws = [semw0, semw1]
    wid = lax.axis_index("s") * 2 + lax.axis_index("c")
    base = pl.multiple_of(wid * _WIDX, 8)
    pltpu.sync_copy(idxv_hbm.at[pl.ds(base, _WIDX)], idxv_v)
    bufs = [bufv.at[0], bufv.at[1]]

    def start_g(n, b):
        io = pl.multiple_of(n * _VROWS, 8)
        pltpu.async_copy(y_hbm.at[idxv_v.at[pl.ds(io, _VROWS)]],
                         bufs[b], semgs[b])

    def wait_g(b):
        pltpu.make_async_copy(y_hbm.at[idxv_v.at[pl.ds(0, _VROWS)]],
                              bufs[b], semgs[b]).wait()

    def start_w(n, b):
        oo = pl.multiple_of(base + n * _VROWS, 8)
        pltpu.async_copy(bufs[b], sv_hbm.at[pl.ds(oo, _VROWS)], semws[b])

    def wait_w(b):
        pltpu.make_async_copy(bufs[b], sv_hbm.at[pl.ds(base, _VROWS)],
                              semws[b]).wait()

    start_g(0, 0)

    def vpair_body(np_, carry):
        for b in range(2):
            n = np_ * 2 + b

            @pl.when((n + 1 < _NVCH) & (n >= 1))
            def _():
                wait_w(1 - b)

            @pl.when(n + 1 < _NVCH)
            def _():
                start_g(n + 1, 1 - b)

            wait_g(b)
            start_w(n, b)
        return carry

    lax.fori_loop(0, _NVCH // 2, vpair_body, 0)
    wait_w(0)
    wait_w(1)


def _sc_gatherv(y_flat, idxv):
    mesh = plsc.VectorSubcoreMesh(core_axis_name="c", subcore_axis_name="s")
    fn = functools.partial(
        pl.kernel,
        out_type=jax.ShapeDtypeStruct((VS * HEADS, EMB), jnp.float32),
        mesh=mesh,
        compiler_params=pltpu.CompilerParams(needs_layout_passes=False),
        scratch_types=[
            pltpu.VMEM((_WIDX,), jnp.int32),
            pltpu.VMEM((2, _VROWS, EMB), jnp.float32),
            pltpu.SemaphoreType.DMA,
            pltpu.SemaphoreType.DMA,
            pltpu.SemaphoreType.DMA,
            pltpu.SemaphoreType.DMA,
        ],
    )(_sc_gatherv_body)
    return fn(y_flat, idxv)


def _sc_dots(y_flat, idxq, idxk):
    mesh = plsc.VectorSubcoreMesh(core_axis_name="c", subcore_axis_name="s")
    fn = functools.partial(
        pl.kernel,
        out_type=jax.ShapeDtypeStruct((HEADS * VS,), jnp.float32),
        mesh=mesh,
        compiler_params=pltpu.CompilerParams(needs_layout_passes=False),
        scratch_types=[
            pltpu.VMEM((_WIDX,), jnp.int32),
            pltpu.VMEM((_WIDX,), jnp.int32),
            pltpu.VMEM((2, 2 * _DCHUNK, EMB), jnp.float32),
            pltpu.VMEM((_DCHUNK * 16,), jnp.float32),
            pltpu.VMEM((_WIDX,), jnp.float32),
            pltpu.SemaphoreType.DMA,
            pltpu.SemaphoreType.DMA,
            pltpu.SemaphoreType.DMA,
            pltpu.SemaphoreType.DMA,
        ],
    )(_sc_dots_body)
    return fn(y_flat, idxq, idxk)


_SM_CHUNK = 512


def _segmax_kernel(dot_ref, w_ref, rows_ref, mx_ref):
    h = pl.program_id(0)
    neg = jnp.float32(-jnp.inf)
    onehot = (lax.broadcasted_iota(jnp.int32, (1, HEADS), 1) == h
              ).astype(jnp.float32)
    laneid = lax.broadcasted_iota(jnp.int32, (1, T), 1)
    run = jnp.full((1, T), neg, jnp.float32)
    for c in range(VS // _SM_CHUNK):
        sl = pl.ds(c * _SM_CHUNK, _SM_CHUNK)
        logit = jnp.sum(dot_ref[sl, :] * onehot, axis=1, keepdims=True)
        logit = logit * w_ref[sl, :]
        mask = rows_ref[sl, :] == laneid
        vals = jnp.where(mask, logit, neg)
        run = jnp.maximum(run, jnp.max(vals, axis=0, keepdims=True))
    mx_ref[...] = run[None]


def _segsm_kernel(mx_ref, dot_ref, w_ref, rows_ref, p_ref):
    rt = (rows_ref[...] == lax.broadcasted_iota(jnp.int32, (1, T), 1)
          ).astype(jnp.float32)
    mrun = mx_ref[...]
    mrun = jnp.where(jnp.isfinite(mrun), mrun, 0.0)
    mx_cand = jnp.dot(rt, mrun, preferred_element_type=jnp.float32)
    logit = dot_ref[...] * w_ref[...]
    ex = jnp.exp(logit - mx_cand)
    srun = lax.dot_general(rt, ex, (((0,), (0,)), ((), ())),
                           preferred_element_type=jnp.float32)
    sm_cand = jnp.dot(rt, srun, preferred_element_type=jnp.float32)
    p_ref[...] = ex / (sm_cand + EPS)


_G_TILE = 256


def _contract_kernel(sv_ref, p_ref, e8_ref, wu_ref, g_ref):
    pexp = jnp.dot(p_ref[...], e8_ref[...], preferred_element_type=jnp.float32)
    g_ref[...] = jnp.dot(sv_ref[...] * pexp, wu_ref[...],
                         preferred_element_type=jnp.float32)


_H_TILE = 256


def _scatter_kernel(rows_ref, g_ref, bu_ref, o_ref):
    m = pl.program_id(0)
    rowiota = lax.broadcasted_iota(jnp.int32, (_H_TILE, 1), 0) + m * _H_TILE
    r = (rowiota == rows_ref[...]).astype(jnp.float32)
    o_ref[...] = jnp.dot(r, g_ref[...],
                         preferred_element_type=jnp.float32) + bu_ref[...]


def kernel(x, means, sigmas, Wq, Wk, Wv, Wu, bu):
    b, t, e = x.shape
    h = HEADS
    x2d = x.reshape(t, e)

    k1, k2 = jax.random.split(jax.random.key(42))
    rr = jax.random.randint(k1, (K, RADD, 2), 0, REGION).astype(jnp.float32)
    rg = jax.random.randint(k2, (K, GADD, 2), 0, t).astype(jnp.float32)

    means2 = means.T.reshape(2, K)
    sig2 = sigmas.reshape(1, K)
    rrx = rr[:, :, 0].T.reshape(RADD, K)
    rry = rr[:, :, 1].T.reshape(RADD, K)
    rgx = rg[:, :, 0].T.reshape(GADD, K)
    rgy = rg[:, :, 1].T.reshape(GADD, K)
    rows20, cols20, mt, st = pl.pallas_call(
        _prep_kernel,
        out_shape=[
            jax.ShapeDtypeStruct((NPTS, K), jnp.int32),
            jax.ShapeDtypeStruct((NPTS, K), jnp.int32),
            jax.ShapeDtypeStruct((2, K), jnp.float32),
            jax.ShapeDtypeStruct((1, K), jnp.float32),
        ],
    )(means2, sig2, rrx, rry, rgx, rgy)
    rows = rows20.T.reshape(VS)
    cols = cols20.T.reshape(VS)

    weights_cm = pl.pallas_call(
        _weights_kernel,
        out_shape=jax.ShapeDtypeStruct((VS, 1), jnp.float32),
    )(rows.reshape(VS, 1), cols.reshape(VS, 1),
      rows.reshape(1, VS), cols.reshape(1, VS), mt, st)

    wqkv = jnp.concatenate([Wq, Wk, Wv], axis=1)
    y = pl.pallas_call(
        _qkv_kernel,
        grid=(3 * h,),
        in_specs=[
            pl.BlockSpec((t, e), lambda j: (0, 0)),
            pl.BlockSpec((e, e), lambda j: (0, j)),
        ],
        out_specs=pl.BlockSpec((1, t, e), lambda j: (j, 0, 0)),
        out_shape=jax.ShapeDtypeStruct((3 * h, t, e), jnp.float32),
    )(x2d, wqkv)
    y_flat = y.reshape(3 * h * t, e)

    hoff = jnp.arange(h, dtype=jnp.int32) * t
    idxq = (rows.reshape(NW, 1, CPW) + hoff[None, :, None]).reshape(-1)
    idxk = (cols.reshape(NW, 1, CPW) + (h * t + hoff)[None, :, None]).reshape(-1)
    idxv = (cols[:, None] + (2 * h * t + hoff)[None, :]).reshape(VS * h)
    dots_wm = _sc_dots(y_flat, idxq, idxk)
    sv = _sc_gatherv(y_flat, idxv)
    dots = dots_wm.reshape(NW, h, CPW).transpose(1, 0, 2).reshape(h, VS)
    sv2d = sv.reshape(VS, h * e)

    dots_cm = dots.T
    rows_cm = rows.reshape(VS, 1)
    mx = pl.pallas_call(
        _segmax_kernel,
        grid=(h,),
        in_specs=[
            pl.BlockSpec((VS, h), lambda hh: (0, 0)),
            pl.BlockSpec((VS, 1), lambda hh: (0, 0)),
            pl.BlockSpec((VS, 1), lambda hh: (0, 0)),
        ],
        out_specs=pl.BlockSpec((1, 1, T), lambda hh: (hh, 0, 0)),
        out_shape=jax.ShapeDtypeStruct((h, 1, T), jnp.float32),
    )(dots_cm, weights_cm, rows_cm)
    p_cm = pl.pallas_call(
        _segsm_kernel,
        out_shape=jax.ShapeDtypeStruct((VS, h), jnp.float32),
    )(mx.reshape(h, T).T, dots_cm, weights_cm, rows_cm)

    e8 = (jnp.repeat(jnp.eye(h, dtype=jnp.float32), e, axis=1)
          )
    g = pl.pallas_call(
        _contract_kernel,
        grid=(VS // _G_TILE,),
        in_specs=[
            pl.BlockSpec((_G_TILE, h * e), lambda m: (m, 0)),
            pl.BlockSpec((_G_TILE, h), lambda m: (m, 0)),
            pl.BlockSpec((h, h * e), lambda m: (0, 0)),
            pl.BlockSpec((h * e, e), lambda m: (0, 0)),
        ],
        out_specs=pl.BlockSpec((_G_TILE, e), lambda m: (m, 0)),
        out_shape=jax.ShapeDtypeStruct((VS, e), jnp.float32),
    )(sv2d, p_cm, e8, Wu)

    out = pl.pallas_call(
        _scatter_kernel,
        grid=(t // _H_TILE,),
        in_specs=[
            pl.BlockSpec((1, VS), lambda m: (0, 0)),
            pl.BlockSpec((VS, e), lambda m: (0, 0)),
            pl.BlockSpec((1, e), lambda m: (0, 0)),
        ],
        out_specs=pl.BlockSpec((_H_TILE, e), lambda m: (m, 0)),
        out_shape=jax.ShapeDtypeStruct((t, e), jnp.float32),
    )(rows.reshape(1, VS), g, bu.reshape(1, e))
    return out.reshape(b, t, e)


# speedup vs baseline: 1.0976x; 1.0009x over previous
import functools

import jax
import jax.numpy as jnp
from jax import lax
from jax.experimental import pallas as pl
from jax.experimental.pallas import tpu as pltpu
from jax.experimental.pallas import tpu_sc as plsc

EMB = 768
HEADS = 8
K = 128
GADD = 8
RADD = 8
REGION = 128
MIN_SIGMA = 0.05
SIGMA_BOOST = 2.0
EPS = 1e-7
T = 2048
NPTS = 4 + RADD + GADD
VS = K * NPTS
SCALE = float(EMB) ** -0.25

NW = 32
CPW = VS // NW


def _prep_kernel(m_ref, s_ref, rrx_ref, rry_ref, rgx_ref, rgy_ref,
                 rows_ref, cols_ref, mt_ref, st_ref):
    t = float(T)
    mtx = jax.nn.sigmoid(m_ref[0:1, :]) * (t - 1.0)
    mty = jax.nn.sigmoid(m_ref[1:2, :]) * (t - 1.0)
    mt_ref[0:1, :] = mtx
    mt_ref[1:2, :] = mty
    s = s_ref[0:1, :] + SIGMA_BOOST
    sp = jnp.maximum(s, 0.0) + jnp.log1p(jnp.exp(-jnp.abs(s)))
    st_ref[0:1, :] = sp + MIN_SIGMA

    flx = jnp.floor(mtx)
    fly = jnp.floor(mty)
    sub4i = lax.broadcasted_iota(jnp.int32, (4, 1), 0)
    offx = (sub4i // 2).astype(jnp.float32)
    offy = (sub4i % 2).astype(jnp.float32)
    cornx = flx + offx
    corny = fly + offy
    lox = jnp.clip(flx - REGION // 2, 0.0, t - REGION)
    loy = jnp.clip(fly - REGION // 2, 0.0, t - REGION)
    relx = lox + rrx_ref[...]
    rely = loy + rry_ref[...]
    globx = rgx_ref[...]
    globy = rgy_ref[...]
    px = jnp.concatenate([cornx, relx, globx], axis=0)
    py = jnp.concatenate([corny, rely, globy], axis=0)
    px = jnp.clip(px, 0.0, t - 1.0)
    py = jnp.clip(py, 0.0, t - 1.0)
    rows_ref[...] = px.astype(jnp.int32)
    cols_ref[...] = py.astype(jnp.int32)


_DUP_CHUNK = 512


def _weights_kernel(rows_cm_ref, cols_cm_ref, rows_rm_ref, cols_rm_ref,
                    mt_ref, st_ref, w_ref):
    t = T
    rows_cm = rows_cm_ref[...]
    cols_cm = cols_cm_ref[...]
    codes_r = rows_cm * t + cols_cm
    codes_c = rows_rm_ref[...] * t + cols_rm_ref[...]
    row_idx = lax.broadcasted_iota(jnp.int32, (VS, 1), 0)
    dup = jnp.zeros((VS, 1), dtype=jnp.bool_)
    for c in range(VS // _DUP_CHUNK):
        cc = codes_c[:, c * _DUP_CHUNK:(c + 1) * _DUP_CHUNK]
        col_idx = lax.broadcasted_iota(jnp.int32, (1, _DUP_CHUNK), 1) + c * _DUP_CHUNK
        eq = (codes_r == cc) & (col_idx < row_idx)
        dup = dup | jnp.any(eq, axis=1, keepdims=True)
    ix = rows_cm.astype(jnp.float32)
    iy = cols_cm.astype(jnp.float32)
    mx = mt_ref[0:1, :]
    my = mt_ref[1:2, :]
    sg = st_ref[0:1, :] + EPS
    dx = (ix - mx) / sg
    dy = (iy - my) / sg
    props = jnp.exp(-0.5 * (dx * dx + dy * dy))
    props = props * (1.0 - dup.astype(jnp.float32))
    colsum = jnp.sum(props, axis=0, keepdims=True)
    props = props / (colsum + EPS)
    w_ref[...] = jnp.sum(props, axis=1, keepdims=True)


def _qkv_kernel(x_ref, w_ref, y_ref):
    j = pl.program_id(0)
    scale = jnp.where(j < 2 * HEADS, SCALE, 1.0)
    y_ref[0] = jnp.dot(x_ref[...], w_ref[...],
                       preferred_element_type=jnp.float32) * scale


_DCHUNK = 32
_NDCH = HEADS * CPW // _DCHUNK
_WIDX = HEADS * CPW
_VROWS = 64
_NVCH = _WIDX // _VROWS


def _sc_dots_body(y_hbm, idxq_hbm, idxk_hbm, dot_hbm,
                  idxq_v, idxk_v, buf, tmp, dloc,
                  semq0, semq1, semk0, semk1):
    semqs = [semq0, semq1]
    semks = [semk0, semk1]
    wid = lax.axis_index("s") * 2 + lax.axis_index("c")
    base = pl.multiple_of(wid * _WIDX, 8)
    laneiota = lax.broadcasted_iota(jnp.int32, (16,), 0)
    pltpu.sync_copy(idxq_hbm.at[pl.ds(base, _WIDX)], idxq_v)
    pltpu.sync_copy(idxk_hbm.at[pl.ds(base, _WIDX)], idxk_v)
    bufsq = [buf.at[0, pl.ds(0, _DCHUNK)], buf.at[1, pl.ds(0, _DCHUNK)]]
    bufsk = [buf.at[0, pl.ds(_DCHUNK, _DCHUNK)],
             buf.at[1, pl.ds(_DCHUNK, _DCHUNK)]]

    def start(n, b):
        io = pl.multiple_of(n * _DCHUNK, 8)
        cq = pltpu.async_copy(y_hbm.at[idxq_v.at[pl.ds(io, _DCHUNK)]],
                              bufsq[b], semqs[b])
        ck = pltpu.async_copy(y_hbm.at[idxk_v.at[pl.ds(io, _DCHUNK)]],
                              bufsk[b], semks[b])
        return cq, ck

    def wait(b):
        pltpu.make_async_copy(y_hbm.at[idxq_v.at[pl.ds(0, _DCHUNK)]],
                              bufsq[b], semqs[b]).wait()
        pltpu.make_async_copy(y_hbm.at[idxk_v.at[pl.ds(0, _DCHUNK)]],
                              bufsk[b], semks[b]).wait()

    start(0, 0)

    def pair_body(np_, carry):
        for b in range(2):
            n = np_ * 2 + b

            @pl.when(n + 1 < _NDCH)
            def _():
                start(n + 1, 1 - b)

            wait(b)
            bq = bufsq[b]
            bk = bufsk[b]
            for g in range(_DCHUNK // 16):

                def row_body(r, rcarry, _g=g):
                    rr = _g * 16 + r
                    acc = bq[rr, pl.ds(0, 16)] * bk[rr, pl.ds(0, 16)]
                    for j in range(1, EMB // 16):
                        acc = acc + bq[rr, pl.ds(j * 16, 16)] * bk[rr, pl.ds(j * 16, 16)]
                    plsc.store_scatter(tmp, [laneiota + r * 16], acc)
                    return rcarry

                lax.fori_loop(0, 16, row_body, 0)
                dots16 = plsc.load_gather(tmp, [laneiota * 16])
                for cc in range(1, 16):
                    dots16 = dots16 + plsc.load_gather(tmp, [laneiota * 16 + cc])
                dloc[pl.ds(pl.multiple_of(n * _DCHUNK + g * 16, 8), 16)] = dots16
        return carry

    lax.fori_loop(0, _NDCH // 2, pair_body, 0)
    pltpu.sync_copy(dloc, dot_hbm.at[pl.ds(base, _WIDX)])


def _sc_gatherv_body(y_hbm, idxv_hbm, sv_hbm, idxv_v, bufv,
                     semg0, semg1, semw0, semw1):
    semgs = [semg0, semg1]
    semws = [semw0, semw1]
    wid = lax.axis_index("s") * 2 + lax.axis_index("c")
    base = pl.multiple_of(wid * _WIDX, 8)
    pltpu.sync_copy(idxv_hbm.at[pl.ds(base, _WIDX)], idxv_v)
    bufs = [bufv.at[0], bufv.at[1]]

    def start_g(n, b):
        io = pl.multiple_of(n * _VROWS, 8)
        pltpu.async_copy(y_hbm.at[idxv_v.at[pl.ds(io, _VROWS)]],
                         bufs[b], semgs[b])

    def wait_g(b):
        pltpu.make_async_copy(y_hbm.at[idxv_v.at[pl.ds(0, _VROWS)]],
                              bufs[b], semgs[b]).wait()

    def start_w(n, b):
        oo = pl.multiple_of(base + n * _VROWS, 8)
        pltpu.async_copy(bufs[b], sv_hbm.at[pl.ds(oo, _VROWS)], semws[b])

    def wait_w(b):
        pltpu.make_async_copy(bufs[b], sv_hbm.at[pl.ds(base, _VROWS)],
                              semws[b]).wait()

    start_g(0, 0)

    def vpair_body(np_, carry):
        for b in range(2):
            n = np_ * 2 + b

            @pl.when((n + 1 < _NVCH) & (n >= 1))
            def _():
                wait_w(1 - b)

            @pl.when(n + 1 < _NVCH)
            def _():
                start_g(n + 1, 1 - b)

            wait_g(b)
            start_w(n, b)
        return carry

    lax.fori_loop(0, _NVCH // 2, vpair_body, 0)
    wait_w(0)
    wait_w(1)


def _sc_gatherv(y_flat, idxv):
    mesh = plsc.VectorSubcoreMesh(core_axis_name="c", subcore_axis_name="s")
    fn = functools.partial(
        pl.kernel,
        out_type=jax.ShapeDtypeStruct((VS * HEADS, EMB), jnp.float32),
        mesh=mesh,
        compiler_params=pltpu.CompilerParams(needs_layout_passes=False),
        scratch_types=[
            pltpu.VMEM((_WIDX,), jnp.int32),
            pltpu.VMEM((2, _VROWS, EMB), jnp.float32),
            pltpu.SemaphoreType.DMA,
            pltpu.SemaphoreType.DMA,
            pltpu.SemaphoreType.DMA,
            pltpu.SemaphoreType.DMA,
        ],
    )(_sc_gatherv_body)
    return fn(y_flat, idxv)


def _sc_dots(y_flat, idxq, idxk):
    mesh = plsc.VectorSubcoreMesh(core_axis_name="c", subcore_axis_name="s")
    fn = functools.partial(
        pl.kernel,
        out_type=jax.ShapeDtypeStruct((HEADS * VS,), jnp.float32),
        mesh=mesh,
        compiler_params=pltpu.CompilerParams(needs_layout_passes=False),
        scratch_types=[
            pltpu.VMEM((_WIDX,), jnp.int32),
            pltpu.VMEM((_WIDX,), jnp.int32),
            pltpu.VMEM((2, 2 * _DCHUNK, EMB), jnp.float32),
            pltpu.VMEM((256,), jnp.float32),
            pltpu.VMEM((_WIDX,), jnp.float32),
            pltpu.SemaphoreType.DMA,
            pltpu.SemaphoreType.DMA,
            pltpu.SemaphoreType.DMA,
            pltpu.SemaphoreType.DMA,
        ],
    )(_sc_dots_body)
    return fn(y_flat, idxq, idxk)


_SM_CHUNK = 512


def _segmax_kernel(dot_ref, w_ref, rows_ref, mx_ref):
    h = pl.program_id(0)
    neg = jnp.float32(-jnp.inf)
    onehot = (lax.broadcasted_iota(jnp.int32, (1, HEADS), 1) == h
              ).astype(jnp.float32)
    laneid = lax.broadcasted_iota(jnp.int32, (1, T), 1)
    run = jnp.full((1, T), neg, jnp.float32)
    for c in range(VS // _SM_CHUNK):
        sl = pl.ds(c * _SM_CHUNK, _SM_CHUNK)
        logit = jnp.sum(dot_ref[sl, :] * onehot, axis=1, keepdims=True)
        logit = logit * w_ref[sl, :]
        mask = rows_ref[sl, :] == laneid
        vals = jnp.where(mask, logit, neg)
        run = jnp.maximum(run, jnp.max(vals, axis=0, keepdims=True))
    mx_ref[...] = run[None]


def _segsm_kernel(mx_ref, dot_ref, w_ref, rows_ref, p_ref):
    rt = (rows_ref[...] == lax.broadcasted_iota(jnp.int32, (1, T), 1)
          ).astype(jnp.float32)
    mrun = mx_ref[...]
    mrun = jnp.where(jnp.isfinite(mrun), mrun, 0.0)
    mx_cand = jnp.dot(rt, mrun, preferred_element_type=jnp.float32)
    logit = dot_ref[...] * w_ref[...]
    ex = jnp.exp(logit - mx_cand)
    srun = lax.dot_general(rt, ex, (((0,), (0,)), ((), ())),
                           preferred_element_type=jnp.float32)
    sm_cand = jnp.dot(rt, srun, preferred_element_type=jnp.float32)
    p_ref[...] = ex / (sm_cand + EPS)


_G_TILE = 256


def _contract_kernel(sv_ref, p_ref, e8_ref, wu_ref, g_ref):
    pexp = jnp.dot(p_ref[...], e8_ref[...], preferred_element_type=jnp.float32)
    g_ref[...] = jnp.dot(sv_ref[...] * pexp, wu_ref[...],
                         preferred_element_type=jnp.float32)


_H_TILE = 256


def _scatter_kernel(rows_ref, g_ref, bu_ref, o_ref):
    m = pl.program_id(0)
    rowiota = lax.broadcasted_iota(jnp.int32, (_H_TILE, 1), 0) + m * _H_TILE
    r = (rowiota == rows_ref[...]).astype(jnp.float32)
    o_ref[...] = jnp.dot(r, g_ref[...],
                         preferred_element_type=jnp.float32) + bu_ref[...]


def kernel(x, means, sigmas, Wq, Wk, Wv, Wu, bu):
    b, t, e = x.shape
    h = HEADS
    x2d = x.reshape(t, e)

    k1, k2 = jax.random.split(jax.random.key(42))
    rr = jax.random.randint(k1, (K, RADD, 2), 0, REGION).astype(jnp.float32)
    rg = jax.random.randint(k2, (K, GADD, 2), 0, t).astype(jnp.float32)

    means2 = means.T.reshape(2, K)
    sig2 = sigmas.reshape(1, K)
    rrx = rr[:, :, 0].T.reshape(RADD, K)
    rry = rr[:, :, 1].T.reshape(RADD, K)
    rgx = rg[:, :, 0].T.reshape(GADD, K)
    rgy = rg[:, :, 1].T.reshape(GADD, K)
    rows20, cols20, mt, st = pl.pallas_call(
        _prep_kernel,
        out_shape=[
            jax.ShapeDtypeStruct((NPTS, K), jnp.int32),
            jax.ShapeDtypeStruct((NPTS, K), jnp.int32),
            jax.ShapeDtypeStruct((2, K), jnp.float32),
            jax.ShapeDtypeStruct((1, K), jnp.float32),
        ],
    )(means2, sig2, rrx, rry, rgx, rgy)
    rows = rows20.T.reshape(VS)
    cols = cols20.T.reshape(VS)

    weights_cm = pl.pallas_call(
        _weights_kernel,
        out_shape=jax.ShapeDtypeStruct((VS, 1), jnp.float32),
    )(rows.reshape(VS, 1), cols.reshape(VS, 1),
      rows.reshape(1, VS), cols.reshape(1, VS), mt, st)

    wqkv = jnp.concatenate([Wq, Wk, Wv], axis=1)
    y = pl.pallas_call(
        _qkv_kernel,
        grid=(3 * h,),
        in_specs=[
            pl.BlockSpec((t, e), lambda j: (0, 0)),
            pl.BlockSpec((e, e), lambda j: (0, j)),
        ],
        out_specs=pl.BlockSpec((1, t, e), lambda j: (j, 0, 0)),
        out_shape=jax.ShapeDtypeStruct((3 * h, t, e), jnp.float32),
    )(x2d, wqkv)
    y_flat = y.reshape(3 * h * t, e)

    hoff = jnp.arange(h, dtype=jnp.int32) * t
    idxq = (rows.reshape(NW, 1, CPW) + hoff[None, :, None]).reshape(-1)
    idxk = (cols.reshape(NW, 1, CPW) + (h * t + hoff)[None, :, None]).reshape(-1)
    idxv = (cols[:, None] + (2 * h * t + hoff)[None, :]).reshape(VS * h)
    dots_wm = _sc_dots(y_flat, idxq, idxk)
    dots = dots_wm.reshape(NW, h, CPW).transpose(1, 0, 2).reshape(h, VS)

    dots_cm = dots.T
    rows_cm = rows.reshape(VS, 1)
    mx = pl.pallas_call(
        _segmax_kernel,
        grid=(h,),
        in_specs=[
            pl.BlockSpec((VS, h), lambda hh: (0, 0)),
            pl.BlockSpec((VS, 1), lambda hh: (0, 0)),
            pl.BlockSpec((VS, 1), lambda hh: (0, 0)),
        ],
        out_specs=pl.BlockSpec((1, 1, T), lambda hh: (hh, 0, 0)),
        out_shape=jax.ShapeDtypeStruct((h, 1, T), jnp.float32),
    )(dots_cm, weights_cm, rows_cm)
    p_cm = pl.pallas_call(
        _segsm_kernel,
        out_shape=jax.ShapeDtypeStruct((VS, h), jnp.float32),
    )(mx.reshape(h, T).T, dots_cm, weights_cm, rows_cm)

    sv = _sc_gatherv(y_flat, idxv)
    sv2d = sv.reshape(VS, h * e)

    e8 = (jnp.repeat(jnp.eye(h, dtype=jnp.float32), e, axis=1)
          )
    g = pl.pallas_call(
        _contract_kernel,
        grid=(VS // _G_TILE,),
        in_specs=[
            pl.BlockSpec((_G_TILE, h * e), lambda m: (m, 0)),
            pl.BlockSpec((_G_TILE, h), lambda m: (m, 0)),
            pl.BlockSpec((h, h * e), lambda m: (0, 0)),
            pl.BlockSpec((h * e, e), lambda m: (0, 0)),
        ],
        out_specs=pl.BlockSpec((_G_TILE, e), lambda m: (m, 0)),
        out_shape=jax.ShapeDtypeStruct((VS, e), jnp.float32),
    )(sv2d, p_cm, e8, Wu)

    out = pl.pallas_call(
        _scatter_kernel,
        grid=(t // _H_TILE,),
        in_specs=[
            pl.BlockSpec((1, VS), lambda m: (0, 0)),
            pl.BlockSpec((VS, e), lambda m: (0, 0)),
            pl.BlockSpec((1, e), lambda m: (0, 0)),
        ],
        out_specs=pl.BlockSpec((_H_TILE, e), lambda m: (m, 0)),
        out_shape=jax.ShapeDtypeStruct((t, e), jnp.float32),
    )(rows.reshape(1, VS), g, bu.reshape(1, e))
    return out.reshape(b, t, e)
